# baseline (speedup 1.0000x reference)
import functools

import jax
import jax.numpy as jnp
from jax import lax
from jax.experimental import pallas as pl
from jax.experimental.pallas import tpu as pltpu
from jax.experimental.pallas import tpu_sc as plsc

_NC = 80
_IOU_T = 0.5
_SCORE_T = 0.5
_LIMIT = 1000
_T = 1024
_NSUB = 16
_LANES = 16
_OUTP = 1024


def _prep_body(geom_ref, s_ref, cls_ref, table_ref):
    cx = geom_ref[0:1, :]
    cy = geom_ref[1:2, :]
    w = geom_ref[2:3, :]
    h = geom_ref[3:4, :]
    s = s_ref[0:1, :]
    x1 = cx - w / 2.0
    y1 = cy - h / 2.0
    x2 = cx + w / 2.0
    y2 = cy + h / 2.0
    mc = jnp.max(jnp.maximum(jnp.maximum(jnp.abs(x1), jnp.abs(y1)),
                             jnp.maximum(jnp.abs(x2), jnp.abs(y2)))) + 1.0
    cls = cls_ref[...]
    m = jnp.max(cls, axis=0, keepdims=True)
    ci = lax.broadcasted_iota(jnp.int32, cls.shape, 0)
    lab = jnp.min(jnp.where(cls == m, ci, _NC), axis=0, keepdims=True)
    labf = lab.astype(jnp.float32)
    off = labf * mc
    xo1 = x1 + off
    yo1 = y1 + off
    xo2 = x2 + off
    yo2 = y2 + off
    area = jnp.maximum(xo2 - xo1, 0.0) * jnp.maximum(yo2 - yo1, 0.0)
    validf = (s > _SCORE_T).astype(jnp.float32)
    z = jnp.zeros_like(s)
    table_ref[...] = jnp.concatenate(
        [xo1, yo1, xo2, yo2, area, x1, y1, x2, y2, s, labf, validf,
         z, z, z, z], axis=0)


def _sc_gather_body(table_hbm, order_hbm, sorted_hbm, idx_v, rows_v, sem,
                    *, npad):
    c = lax.axis_index("c")
    s = lax.axis_index("s")
    rpw = npad // _NSUB
    nch = rpw // 64
    base = c * npad + s * rpw
    lcps = [pltpu.async_copy(order_hbm.at[pl.ds(base + j * 64, 64)],
                             idx_v.at[j], sem) for j in range(nch)]
    for cp in lcps:
        cp.wait()
    ibase = jnp.zeros((_LANES,), jnp.int32) + c * npad
    for j in range(nch):
        for i in range(64 // _LANES):
            sl = pl.ds(i * _LANES, _LANES)
            idx_v[j, sl] = idx_v[j, sl] + ibase
    gcps = [pltpu.async_copy(table_hbm.at[idx_v.at[j]],
                             rows_v.at[pl.ds(j * 64, 64)], sem)
            for j in range(nch)]
    for cp in gcps:
        cp.wait()
    pltpu.sync_copy(rows_v, sorted_hbm.at[pl.ds(base, rpw)])


def _sc_scatter_body(rows_hbm, sidx_hbm, out_hbm, idx_v, rows_v, z_v, sem,
                     sem2, *, npad):
    c = lax.axis_index("c")
    s = lax.axis_index("s")
    rpw = npad // _NSUB
    nch = rpw // 64
    base = c * npad + s * rpw
    zrows = _OUTP // _NSUB
    for i in range(zrows):
        for k in range(128 // _LANES):
            z_v[i, pl.ds(k * _LANES, _LANES)] = jnp.zeros((_LANES,),
                                                          jnp.float32)
    zcp = pltpu.async_copy(z_v, out_hbm.at[pl.ds(c * _OUTP + s * zrows,
                                                 zrows)], sem)
    lcps = [pltpu.async_copy(sidx_hbm.at[pl.ds(base + j * 64, 64)],
                             idx_v.at[j], sem2) for j in range(nch)]
    rcp = pltpu.async_copy(rows_hbm.at[pl.ds(base, rpw)], rows_v, sem2)
    zcp.wait()
    plsc.subcore_barrier()
    for cp in lcps:
        cp.wait()
    rcp.wait()
    scps = [pltpu.async_copy(rows_v.at[pl.ds(j * 64, 64)],
                             out_hbm.at[idx_v.at[j]], sem)
            for j in range(nch)]
    for cp in scps:
        cp.wait()


def _nms_body(sorted3_ref, sortedt3_ref, sidx_ref, act_ref, *, nt):
    t_ = _T
    act_ref[...] = sortedt3_ref[:, 11:12, :]
    nv = jnp.sum(act_ref[...])
    nta = jnp.minimum(jnp.ceil(nv / t_), float(nt)).astype(jnp.int32)

    ii = lax.broadcasted_iota(jnp.int32, (t_, t_), 0)
    jj = lax.broadcasted_iota(jnp.int32, (t_, t_), 1)
    tri = ii < jj
    dn = (((1,), (0,)), ((), ()))
    obase = pl.program_id(0) * _OUTP
    lane = lax.broadcasted_iota(jnp.int32, (1, t_), 1)
    dummy = _LIMIT + lax.rem(lane, _NSUB) + obase
    sidx_ref[...] = jnp.broadcast_to(dummy, (nt, 1, t_))

    def get_cols(t):
        st = sorted3_ref[t]
        return (st[:, 0:1], st[:, 1:2], st[:, 2:3], st[:, 3:4], st[:, 4:5])

    def get_rows(k):
        rt = sortedt3_ref[k]
        return (rt[0:1, :], rt[1:2, :], rt[2:3, :], rt[3:4, :], rt[4:5, :])

    def overlaps(c, r):
        ltx = jnp.maximum(c[0], r[0])
        lty = jnp.maximum(c[1], r[1])
        rbx = jnp.minimum(c[2], r[2])
        rby = jnp.minimum(c[3], r[3])
        whx = jnp.maximum(rbx - ltx, 0.0)
        why = jnp.maximum(rby - lty, 0.0)
        inter = whx * why
        un = (c[4] + r[4]) - inter
        return inter > _IOU_T * un

    def tile_body(t, carry):
        c = get_cols(t)
        sf = jnp.where(overlaps(c, get_rows(t)) & tri, 1.0, 0.0)
        a0 = act_ref[t]

        def cond(cr):
            return cr[1]

        def fbody(cr):
            a, _ = cr
            sup = lax.dot_general(a, sf, dn,
                                  preferred_element_type=jnp.float32)
            anew = jnp.where(sup > 0.0, 0.0, a0)
            return (anew, jnp.any(anew != a))

        a_fin, _ = lax.while_loop(cond, fbody, (a0, True))
        act_ref[t] = a_fin

        def cross(k, cc):
            sc = jnp.where(overlaps(c, get_rows(k)), 1.0, 0.0)
            sup = lax.dot_general(a_fin, sc, dn,
                                  preferred_element_type=jnp.float32)
            act_ref[k] = jnp.where(sup > 0.0, 0.0, act_ref[k])
            return cc

        @pl.when(jnp.sum(a_fin) > 0.0)
        def _():
            lax.fori_loop(t + 1, nta, cross, 0)

        return carry

    lax.fori_loop(0, nta, tile_body, 0)

    u = jnp.where(tri, 1.0, 0.0)

    def pos_body(t, cnt):
        krow = act_ref[t]
        excl = lax.dot_general(krow, u, dn,
                               preferred_element_type=jnp.float32) + cnt
        ok = (krow > 0.0) & (excl < float(_LIMIT))
        sidx_ref[t] = jnp.where(ok, excl.astype(jnp.int32) + obase, dummy)
        return cnt + jnp.sum(krow)

    lax.fori_loop(0, nta, pos_body, 0.0)


def kernel(yolo_batch):
    b, n, _ = yolo_batch.shape
    npad = ((n + _T - 1) // _T) * _T
    nt = npad // _T
    yp = jnp.pad(yolo_batch, ((0, 0), (0, npad - n), (0, 0)))
    s = jax.nn.sigmoid(yp[..., 4])
    skey = jnp.where(s > _SCORE_T, s, -jnp.inf)
    order = jnp.argsort(-skey, axis=-1).astype(jnp.int32)

    geomt = jnp.swapaxes(yp[..., 0:4], 1, 2)
    srow = s[:, None, :]
    clst = jnp.swapaxes(yp[..., 5:5 + _NC], 1, 2)

    tablet = pl.pallas_call(
        _prep_body,
        grid=(b,),
        in_specs=[pl.BlockSpec((None, 4, npad), lambda i: (i, 0, 0)),
                  pl.BlockSpec((None, 1, npad), lambda i: (i, 0, 0)),
                  pl.BlockSpec((None, _NC, npad), lambda i: (i, 0, 0))],
        out_specs=pl.BlockSpec((None, 16, npad), lambda i: (i, 0, 0)),
        out_shape=jax.ShapeDtypeStruct((b, 16, npad), jnp.float32),
    )(geomt, srow, clst)

    table_flat = jnp.pad(jnp.swapaxes(tablet, 1, 2).reshape(b * npad, 16),
                         ((0, 0), (0, 112)))
    order1d = order.reshape(b * npad)

    mesh = plsc.VectorSubcoreMesh(core_axis_name="c", subcore_axis_name="s")
    rpw = npad // _NSUB
    sorted_flat = pl.kernel(
        functools.partial(_sc_gather_body, npad=npad),
        out_type=jax.ShapeDtypeStruct((b * npad, 128), jnp.float32),
        mesh=mesh,
        scratch_types=[pltpu.VMEM((rpw // 64, 64), jnp.int32),
                       pltpu.VMEM((rpw, 128), jnp.float32),
                       pltpu.SemaphoreType.DMA],
    )(table_flat, order1d)

    sorted3 = sorted_flat[:, :16].reshape(b, nt, _T, 16)
    sortedt3 = jnp.swapaxes(sorted3, 2, 3)

    sidx = pl.pallas_call(
        functools.partial(_nms_body, nt=nt),
        grid=(b,),
        in_specs=[pl.BlockSpec((None, nt, _T, 16), lambda i: (i, 0, 0, 0)),
                  pl.BlockSpec((None, nt, 16, _T), lambda i: (i, 0, 0, 0))],
        out_specs=pl.BlockSpec((None, nt, 1, _T), lambda i: (i, 0, 0, 0)),
        out_shape=jax.ShapeDtypeStruct((b, nt, 1, _T), jnp.int32),
        scratch_shapes=[pltpu.VMEM((nt, 1, _T), jnp.float32)],
    )(sorted3, sortedt3)

    sidx1d = sidx.reshape(b * npad)
    out_flat = pl.kernel(
        functools.partial(_sc_scatter_body, npad=npad),
        out_type=jax.ShapeDtypeStruct((b * _OUTP, 128), jnp.float32),
        mesh=mesh,
        scratch_types=[pltpu.VMEM((rpw // 64, 64), jnp.int32),
                       pltpu.VMEM((rpw, 128), jnp.float32),
                       pltpu.VMEM((_OUTP // _NSUB, 128), jnp.float32),
                       pltpu.SemaphoreType.DMA,
                       pltpu.SemaphoreType.DMA],
    )(sorted_flat, sidx1d)

    return out_flat.reshape(b, _OUTP, 128)[:, :_LIMIT, 5:11]


# speedup vs baseline: 1.0489x; 1.0489x over previous
import functools

import jax
import jax.numpy as jnp
from jax import lax
from jax.experimental import pallas as pl
from jax.experimental.pallas import tpu as pltpu
from jax.experimental.pallas import tpu_sc as plsc

_NC = 80
_IOU_T = 0.5
_SCORE_T = 0.5
_LIMIT = 1000
_T = 640
_NSUB = 16
_LANES = 16
_OUTP = 1024


def _prep_body(geom_ref, s_ref, cls_ref, table_ref):
    cx = geom_ref[0:1, :]
    cy = geom_ref[1:2, :]
    w = geom_ref[2:3, :]
    h = geom_ref[3:4, :]
    s = s_ref[0:1, :]
    x1 = cx - w / 2.0
    y1 = cy - h / 2.0
    x2 = cx + w / 2.0
    y2 = cy + h / 2.0
    mc = jnp.max(jnp.maximum(jnp.maximum(jnp.abs(x1), jnp.abs(y1)),
                             jnp.maximum(jnp.abs(x2), jnp.abs(y2)))) + 1.0
    cls = cls_ref[...]
    m = jnp.max(cls, axis=0, keepdims=True)
    ci = lax.broadcasted_iota(jnp.int32, cls.shape, 0)
    lab = jnp.min(jnp.where(cls == m, ci, _NC), axis=0, keepdims=True)
    labf = lab.astype(jnp.float32)
    off = labf * mc
    xo1 = x1 + off
    yo1 = y1 + off
    xo2 = x2 + off
    yo2 = y2 + off
    area = jnp.maximum(xo2 - xo1, 0.0) * jnp.maximum(yo2 - yo1, 0.0)
    validf = (s > _SCORE_T).astype(jnp.float32)
    z = jnp.zeros_like(s)
    table_ref[...] = jnp.concatenate(
        [xo1, yo1, xo2, yo2, area, x1, y1, x2, y2, s, labf, validf,
         z, z, z, z], axis=0)


def _sc_gather_body(table_hbm, order_hbm, sorted_hbm, idx_v, rows_v, sem,
                    *, npad):
    c = lax.axis_index("c")
    s = lax.axis_index("s")
    rpw = npad // _NSUB
    nch = rpw // 64
    base = c * npad + s * rpw
    lcps = [pltpu.async_copy(order_hbm.at[pl.ds(base + j * 64, 64)],
                             idx_v.at[j], sem) for j in range(nch)]
    for cp in lcps:
        cp.wait()
    ibase = jnp.zeros((_LANES,), jnp.int32) + c * npad
    for j in range(nch):
        for i in range(64 // _LANES):
            sl = pl.ds(i * _LANES, _LANES)
            idx_v[j, sl] = idx_v[j, sl] + ibase
    gcps = [pltpu.async_copy(table_hbm.at[idx_v.at[j]],
                             rows_v.at[pl.ds(j * 64, 64)], sem)
            for j in range(nch)]
    for cp in gcps:
        cp.wait()
    pltpu.sync_copy(rows_v, sorted_hbm.at[pl.ds(base, rpw)])


def _sc_scatter_body(rows_hbm, sidx_hbm, out_hbm, idx_v, rows_v, z_v, sem,
                     sem2, *, npad):
    c = lax.axis_index("c")
    s = lax.axis_index("s")
    rpw = npad // _NSUB
    nch = rpw // 64
    base = c * npad + s * rpw
    zrows = _OUTP // _NSUB
    for i in range(zrows):
        for k in range(128 // _LANES):
            z_v[i, pl.ds(k * _LANES, _LANES)] = jnp.zeros((_LANES,),
                                                          jnp.float32)
    zcp = pltpu.async_copy(z_v, out_hbm.at[pl.ds(c * _OUTP + s * zrows,
                                                 zrows)], sem)
    lcps = [pltpu.async_copy(sidx_hbm.at[pl.ds(base + j * 64, 64)],
                             idx_v.at[j], sem2) for j in range(nch)]
    rcp = pltpu.async_copy(rows_hbm.at[pl.ds(base, rpw)], rows_v, sem2)
    zcp.wait()
    plsc.subcore_barrier()
    for cp in lcps:
        cp.wait()
    rcp.wait()
    scps = [pltpu.async_copy(rows_v.at[pl.ds(j * 64, 64)],
                             out_hbm.at[idx_v.at[j]], sem)
            for j in range(nch)]
    for cp in scps:
        cp.wait()


def _nms_body(sorted3_ref, sortedt3_ref, sidx_ref, act_ref, *, nt):
    t_ = _T
    act_ref[...] = sortedt3_ref[:, 11:12, :]
    nv = jnp.sum(act_ref[...])
    nta = jnp.minimum(jnp.ceil(nv / t_), float(nt)).astype(jnp.int32)

    ii = lax.broadcasted_iota(jnp.int32, (t_, t_), 0)
    jj = lax.broadcasted_iota(jnp.int32, (t_, t_), 1)
    tri = ii < jj
    dn = (((1,), (0,)), ((), ()))
    obase = pl.program_id(0) * _OUTP
    lane = lax.broadcasted_iota(jnp.int32, (1, t_), 1)
    dummy = _LIMIT + lax.rem(lane, _NSUB) + obase
    sidx_ref[...] = jnp.broadcast_to(dummy, (nt, 1, t_))

    def get_cols(t):
        st = sorted3_ref[t]
        return (st[:, 0:1], st[:, 1:2], st[:, 2:3], st[:, 3:4], st[:, 4:5])

    def get_rows(k):
        rt = sortedt3_ref[k]
        return (rt[0:1, :], rt[1:2, :], rt[2:3, :], rt[3:4, :], rt[4:5, :])

    def overlaps(c, r):
        ltx = jnp.maximum(c[0], r[0])
        lty = jnp.maximum(c[1], r[1])
        rbx = jnp.minimum(c[2], r[2])
        rby = jnp.minimum(c[3], r[3])
        whx = jnp.maximum(rbx - ltx, 0.0)
        why = jnp.maximum(rby - lty, 0.0)
        inter = whx * why
        un = (c[4] + r[4]) - inter
        return inter > _IOU_T * un

    def tile_body(t, carry):
        c = get_cols(t)
        sf = jnp.where(overlaps(c, get_rows(t)) & tri, 1.0, 0.0)
        a0 = act_ref[t]

        def cond(cr):
            return cr[1]

        def fbody(cr):
            a, _ = cr
            sup = lax.dot_general(a, sf, dn,
                                  preferred_element_type=jnp.float32)
            anew = jnp.where(sup > 0.0, 0.0, a0)
            return (anew, jnp.any(anew != a))

        a_fin, _ = lax.while_loop(cond, fbody, (a0, True))
        act_ref[t] = a_fin

        def cross(k, cc):
            sc = jnp.where(overlaps(c, get_rows(k)), 1.0, 0.0)
            sup = lax.dot_general(a_fin, sc, dn,
                                  preferred_element_type=jnp.float32)
            act_ref[k] = jnp.where(sup > 0.0, 0.0, act_ref[k])
            return cc

        @pl.when(jnp.sum(a_fin) > 0.0)
        def _():
            lax.fori_loop(t + 1, nta, cross, 0)

        return carry

    lax.fori_loop(0, nta, tile_body, 0)

    u = jnp.where(tri, 1.0, 0.0)

    def pos_body(t, cnt):
        krow = act_ref[t]
        excl = lax.dot_general(krow, u, dn,
                               preferred_element_type=jnp.float32) + cnt
        ok = (krow > 0.0) & (excl < float(_LIMIT))
        sidx_ref[t] = jnp.where(ok, excl.astype(jnp.int32) + obase, dummy)
        return cnt + jnp.sum(krow)

    lax.fori_loop(0, nta, pos_body, 0.0)


def kernel(yolo_batch):
    b, n, _ = yolo_batch.shape
    npad = ((n + _T - 1) // _T) * _T
    nt = npad // _T
    yp = jnp.pad(yolo_batch, ((0, 0), (0, npad - n), (0, 0)))
    s = jax.nn.sigmoid(yp[..., 4])
    skey = jnp.where(s > _SCORE_T, s, -jnp.inf)
    order = jnp.argsort(-skey, axis=-1).astype(jnp.int32)

    geomt = jnp.swapaxes(yp[..., 0:4], 1, 2)
    srow = s[:, None, :]
    clst = jnp.swapaxes(yp[..., 5:5 + _NC], 1, 2)

    tablet = pl.pallas_call(
        _prep_body,
        grid=(b,),
        in_specs=[pl.BlockSpec((None, 4, npad), lambda i: (i, 0, 0)),
                  pl.BlockSpec((None, 1, npad), lambda i: (i, 0, 0)),
                  pl.BlockSpec((None, _NC, npad), lambda i: (i, 0, 0))],
        out_specs=pl.BlockSpec((None, 16, npad), lambda i: (i, 0, 0)),
        out_shape=jax.ShapeDtypeStruct((b, 16, npad), jnp.float32),
    )(geomt, srow, clst)

    table_flat = jnp.pad(jnp.swapaxes(tablet, 1, 2).reshape(b * npad, 16),
                         ((0, 0), (0, 112)))
    order1d = order.reshape(b * npad)

    mesh = plsc.VectorSubcoreMesh(core_axis_name="c", subcore_axis_name="s")
    rpw = npad // _NSUB
    sorted_flat = pl.kernel(
        functools.partial(_sc_gather_body, npad=npad),
        out_type=jax.ShapeDtypeStruct((b * npad, 128), jnp.float32),
        mesh=mesh,
        scratch_types=[pltpu.VMEM((rpw // 64, 64), jnp.int32),
                       pltpu.VMEM((rpw, 128), jnp.float32),
                       pltpu.SemaphoreType.DMA],
    )(table_flat, order1d)

    sorted3 = sorted_flat[:, :16].reshape(b, nt, _T, 16)
    sortedt3 = jnp.swapaxes(sorted3, 2, 3)

    sidx = pl.pallas_call(
        functools.partial(_nms_body, nt=nt),
        grid=(b,),
        in_specs=[pl.BlockSpec((None, nt, _T, 16), lambda i: (i, 0, 0, 0)),
                  pl.BlockSpec((None, nt, 16, _T), lambda i: (i, 0, 0, 0))],
        out_specs=pl.BlockSpec((None, nt, 1, _T), lambda i: (i, 0, 0, 0)),
        out_shape=jax.ShapeDtypeStruct((b, nt, 1, _T), jnp.int32),
        scratch_shapes=[pltpu.VMEM((nt, 1, _T), jnp.float32)],
    )(sorted3, sortedt3)

    sidx1d = sidx.reshape(b * npad)
    out_flat = pl.kernel(
        functools.partial(_sc_scatter_body, npad=npad),
        out_type=jax.ShapeDtypeStruct((b * _OUTP, 128), jnp.float32),
        mesh=mesh,
        scratch_types=[pltpu.VMEM((rpw // 64, 64), jnp.int32),
                       pltpu.VMEM((rpw, 128), jnp.float32),
                       pltpu.VMEM((_OUTP // _NSUB, 128), jnp.float32),
                       pltpu.SemaphoreType.DMA,
                       pltpu.SemaphoreType.DMA],
    )(sorted_flat, sidx1d)

    return out_flat.reshape(b, _OUTP, 128)[:, :_LIMIT, 5:11]


# speedup vs baseline: 1.0520x; 1.0030x over previous
import functools

import jax
import jax.numpy as jnp
from jax import lax
from jax.experimental import pallas as pl
from jax.experimental.pallas import tpu as pltpu
from jax.experimental.pallas import tpu_sc as plsc

_NC = 80
_IOU_T = 0.5
_SCORE_T = 0.5
_LIMIT = 1000
_T = 640
_NSUB = 16
_LANES = 16
_OUTP = 1024


def _prep_body(geom_ref, s_ref, cls_ref, table_ref):
    cx = geom_ref[0:1, :]
    cy = geom_ref[1:2, :]
    w = geom_ref[2:3, :]
    h = geom_ref[3:4, :]
    s = s_ref[0:1, :]
    x1 = cx - w / 2.0
    y1 = cy - h / 2.0
    x2 = cx + w / 2.0
    y2 = cy + h / 2.0
    mc = jnp.max(jnp.maximum(jnp.maximum(jnp.abs(x1), jnp.abs(y1)),
                             jnp.maximum(jnp.abs(x2), jnp.abs(y2)))) + 1.0
    cls = cls_ref[...]
    m = jnp.max(cls, axis=0, keepdims=True)
    ci = lax.broadcasted_iota(jnp.int32, cls.shape, 0)
    lab = jnp.min(jnp.where(cls == m, ci, _NC), axis=0, keepdims=True)
    labf = lab.astype(jnp.float32)
    off = labf * mc
    xo1 = x1 + off
    yo1 = y1 + off
    xo2 = x2 + off
    yo2 = y2 + off
    area = jnp.maximum(xo2 - xo1, 0.0) * jnp.maximum(yo2 - yo1, 0.0)
    validf = (s > _SCORE_T).astype(jnp.float32)
    z = jnp.zeros_like(s)
    table_ref[...] = jnp.concatenate(
        [xo1, yo1, xo2, yo2, area, x1, y1, x2, y2, s, labf, validf,
         z, z, z, z], axis=0)


def _sc_gather_body(table_hbm, order_hbm, sorted_hbm, idx_v, rows_v, sem,
                    *, npad):
    c = lax.axis_index("c")
    s = lax.axis_index("s")
    rpw = npad // _NSUB
    nch = rpw // 64
    base = c * npad + s * rpw
    lcps = [pltpu.async_copy(order_hbm.at[pl.ds(base + j * 64, 64)],
                             idx_v.at[j], sem) for j in range(nch)]
    for cp in lcps:
        cp.wait()
    ibase = jnp.zeros((_LANES,), jnp.int32) + c * npad
    for j in range(nch):
        for i in range(64 // _LANES):
            sl = pl.ds(i * _LANES, _LANES)
            idx_v[j, sl] = idx_v[j, sl] + ibase
    gcps = [pltpu.async_copy(table_hbm.at[idx_v.at[j]],
                             rows_v.at[pl.ds(j * 64, 64)], sem)
            for j in range(nch)]
    for cp in gcps:
        cp.wait()
    pltpu.sync_copy(rows_v, sorted_hbm.at[pl.ds(base, rpw)])


def _sc_scatter_body(rows_hbm, sidx_hbm, out_hbm, idx_v, rows_v, z_v, sem,
                     sem2, *, npad):
    c = lax.axis_index("c")
    s = lax.axis_index("s")
    rpw = npad // _NSUB
    nch = rpw // 64
    base = c * npad + s * rpw
    zrows = _OUTP // _NSUB
    lcps = [pltpu.async_copy(sidx_hbm.at[pl.ds(base + j * 64, 64)],
                             idx_v.at[j], sem2) for j in range(nch)]
    rcp = pltpu.async_copy(rows_hbm.at[pl.ds(base, rpw)], rows_v, sem2)
    for i in range(zrows):
        for k in range(128 // _LANES):
            z_v[i, pl.ds(k * _LANES, _LANES)] = jnp.zeros((_LANES,),
                                                          jnp.float32)
    zcp = pltpu.async_copy(z_v, out_hbm.at[pl.ds(c * _OUTP + s * zrows,
                                                 zrows)], sem)
    zcp.wait()
    plsc.subcore_barrier()
    for cp in lcps:
        cp.wait()
    rcp.wait()
    scps = [pltpu.async_copy(rows_v.at[pl.ds(j * 64, 64)],
                             out_hbm.at[idx_v.at[j]], sem)
            for j in range(nch)]
    for cp in scps:
        cp.wait()


def _nms_body(sorted3_ref, sortedt3_ref, sidx_ref, act_ref, *, nt):
    t_ = _T
    act_ref[...] = sortedt3_ref[:, 11:12, :]
    nv = jnp.sum(act_ref[...])
    nta = jnp.minimum(jnp.ceil(nv / t_), float(nt)).astype(jnp.int32)

    ii = lax.broadcasted_iota(jnp.int32, (t_, t_), 0)
    jj = lax.broadcasted_iota(jnp.int32, (t_, t_), 1)
    tri = ii < jj
    dn = (((1,), (0,)), ((), ()))
    obase = pl.program_id(0) * _OUTP
    lane = lax.broadcasted_iota(jnp.int32, (1, t_), 1)
    dummy = _LIMIT + lax.rem(lane, _NSUB) + obase
    sidx_ref[...] = jnp.broadcast_to(dummy, (nt, 1, t_))

    def get_cols(t):
        st = sorted3_ref[t]
        return (st[:, 0:1], st[:, 1:2], st[:, 2:3], st[:, 3:4], st[:, 4:5])

    def get_rows(k):
        rt = sortedt3_ref[k]
        return (rt[0:1, :], rt[1:2, :], rt[2:3, :], rt[3:4, :], rt[4:5, :])

    def overlaps(c, r):
        ltx = jnp.maximum(c[0], r[0])
        lty = jnp.maximum(c[1], r[1])
        rbx = jnp.minimum(c[2], r[2])
        rby = jnp.minimum(c[3], r[3])
        whx = jnp.maximum(rbx - ltx, 0.0)
        why = jnp.maximum(rby - lty, 0.0)
        inter = whx * why
        un = (c[4] + r[4]) - inter
        return inter > _IOU_T * un

    def tile_body(t, carry):
        c = get_cols(t)
        sf = jnp.where(overlaps(c, get_rows(t)) & tri, 1.0, 0.0)
        a0 = act_ref[t]

        def cond(cr):
            return cr[1]

        def fbody(cr):
            a, _ = cr
            sup = lax.dot_general(a, sf, dn,
                                  preferred_element_type=jnp.float32)
            anew = jnp.where(sup > 0.0, 0.0, a0)
            return (anew, jnp.any(anew != a))

        a_fin, _ = lax.while_loop(cond, fbody, (a0, True))
        act_ref[t] = a_fin

        def cross(k, cc):
            sc = jnp.where(overlaps(c, get_rows(k)), 1.0, 0.0)
            sup = lax.dot_general(a_fin, sc, dn,
                                  preferred_element_type=jnp.float32)
            act_ref[k] = jnp.where(sup > 0.0, 0.0, act_ref[k])
            return cc

        @pl.when(jnp.sum(a_fin) > 0.0)
        def _():
            lax.fori_loop(t + 1, nta, cross, 0)

        return carry

    lax.fori_loop(0, nta, tile_body, 0)

    u = jnp.where(tri, 1.0, 0.0)

    def pos_body(t, cnt):
        krow = act_ref[t]
        excl = lax.dot_general(krow, u, dn,
                               preferred_element_type=jnp.float32) + cnt
        ok = (krow > 0.0) & (excl < float(_LIMIT))
        sidx_ref[t] = jnp.where(ok, excl.astype(jnp.int32) + obase, dummy)
        return cnt + jnp.sum(krow)

    lax.fori_loop(0, nta, pos_body, 0.0)


def kernel(yolo_batch):
    b, n, _ = yolo_batch.shape
    npad = ((n + _T - 1) // _T) * _T
    nt = npad // _T
    yp = jnp.pad(yolo_batch, ((0, 0), (0, npad - n), (0, 0)))
    s = jax.nn.sigmoid(yp[..., 4])
    skey = jnp.where(s > _SCORE_T, s, -jnp.inf)
    order = jnp.argsort(-skey, axis=-1).astype(jnp.int32)

    geomt = jnp.swapaxes(yp[..., 0:4], 1, 2)
    srow = s[:, None, :]
    clst = jnp.swapaxes(yp[..., 5:5 + _NC], 1, 2)

    tablet = pl.pallas_call(
        _prep_body,
        grid=(b,),
        in_specs=[pl.BlockSpec((None, 4, npad), lambda i: (i, 0, 0)),
                  pl.BlockSpec((None, 1, npad), lambda i: (i, 0, 0)),
                  pl.BlockSpec((None, _NC, npad), lambda i: (i, 0, 0))],
        out_specs=pl.BlockSpec((None, 16, npad), lambda i: (i, 0, 0)),
        out_shape=jax.ShapeDtypeStruct((b, 16, npad), jnp.float32),
    )(geomt, srow, clst)

    table_flat = jnp.pad(jnp.swapaxes(tablet, 1, 2).reshape(b * npad, 16),
                         ((0, 0), (0, 112)))
    order1d = order.reshape(b * npad)

    mesh = plsc.VectorSubcoreMesh(core_axis_name="c", subcore_axis_name="s")
    rpw = npad // _NSUB
    sorted_flat = pl.kernel(
        functools.partial(_sc_gather_body, npad=npad),
        out_type=jax.ShapeDtypeStruct((b * npad, 128), jnp.float32),
        mesh=mesh,
        scratch_types=[pltpu.VMEM((rpw // 64, 64), jnp.int32),
                       pltpu.VMEM((rpw, 128), jnp.float32),
                       pltpu.SemaphoreType.DMA],
    )(table_flat, order1d)

    sorted3 = sorted_flat[:, :16].reshape(b, nt, _T, 16)
    sortedt3 = jnp.swapaxes(sorted3, 2, 3)

    sidx = pl.pallas_call(
        functools.partial(_nms_body, nt=nt),
        grid=(b,),
        in_specs=[pl.BlockSpec((None, nt, _T, 16), lambda i: (i, 0, 0, 0)),
                  pl.BlockSpec((None, nt, 16, _T), lambda i: (i, 0, 0, 0))],
        out_specs=pl.BlockSpec((None, nt, 1, _T), lambda i: (i, 0, 0, 0)),
        out_shape=jax.ShapeDtypeStruct((b, nt, 1, _T), jnp.int32),
        scratch_shapes=[pltpu.VMEM((nt, 1, _T), jnp.float32)],
    )(sorted3, sortedt3)

    sidx1d = sidx.reshape(b * npad)
    out_flat = pl.kernel(
        functools.partial(_sc_scatter_body, npad=npad),
        out_type=jax.ShapeDtypeStruct((b * _OUTP, 128), jnp.float32),
        mesh=mesh,
        scratch_types=[pltpu.VMEM((rpw // 64, 64), jnp.int32),
                       pltpu.VMEM((rpw, 128), jnp.float32),
                       pltpu.VMEM((_OUTP // _NSUB, 128), jnp.float32),
                       pltpu.SemaphoreType.DMA,
                       pltpu.SemaphoreType.DMA],
    )(sorted_flat, sidx1d)

    return out_flat.reshape(b, _OUTP, 128)[:, :_LIMIT, 5:11]
